# Initial kernel scaffold; baseline (speedup 1.0000x reference)
import jax
import jax.numpy as jnp
from jax.experimental import pallas as pl


def kernel(h, edge_index, e, WA, bA, WB, bB, WC, bC, WD, bD, WE, bE, gamma_h, beta_h, gamma_e, beta_e):
    raise NotImplementedError("write your pallas kernel here")


# speedup vs baseline: 1.5883x; 1.5883x over previous
import functools

import jax
import jax.numpy as jnp
from jax import lax
from jax.experimental import pallas as pl
from jax.experimental.pallas import tpu as pltpu
from jax.experimental.pallas import tpu_sc as plsc

_NC = 2
_NS = 16
_L = 16
_C = 80


def _proj_h_body(h_ref, wb, wd, we, bb, bd, be_, bhc, dhc, ehc):
    hb = h_ref[...]
    bhc[...] = jnp.dot(hb, wb[0], preferred_element_type=jnp.float32) + bb[0]
    dhc[...] = jnp.dot(hb, wd[0], preferred_element_type=jnp.float32) + bd[0]
    ehc[...] = jnp.dot(hb, we[0], preferred_element_type=jnp.float32) + be_[0]


def _proj_e_body(e_ref, wc, bc, cec):
    cec[...] = (jnp.dot(e_ref[...], wc[0], preferred_element_type=jnp.float32)
                + bc[0])


def _sc_edge_body(n_nodes, n_edges, hh,
                  src_hbm, dst_hbm, dhc_hbm, ehc_hbm, bhc_hbm, cec_hbm,
                  eij_hbm, num_hbm, den_hbm, stats_hbm,
                  src_v, dst_v, srcT_v, dstT_v,
                  dh_v, eh_v, bh_v, ce_v, sg_v, sb_v,
                  st_v, zb_v, num_s, den_s,
                  sem1, sem2, sem3):
    c = lax.axis_index("c")
    s = lax.axis_index("s")
    G = hh // _L

    zero = jnp.zeros((_L,), jnp.float32)

    def zb_row(i, carry):
        for j in range(G):
            zb_v[i, pl.ds(j * _L, _L)] = zero
        return carry
    lax.fori_loop(0, _C, zb_row, 0)

    for j in range(2 * G):
        st_v[pl.ds(j * _L, _L)] = zero

    rpt = (-(-n_nodes // _NS) + _C - 1) // _C * _C
    base0 = s * rpt
    nrows = jnp.maximum(0, jnp.minimum(rpt, n_nodes - base0))
    ncopy = nrows // _C

    def z_copy(k, carry):
        off = base0 + k * _C
        pltpu.sync_copy(zb_v, num_s.at[pl.ds(off, _C)])
        pltpu.sync_copy(zb_v, den_s.at[pl.ds(off, _C)])
        return carry
    lax.fori_loop(0, ncopy, z_copy, 0)
    plsc.subcore_barrier()

    et = n_edges // _NS
    nch = et // _C
    t0 = s * et
    coff = c * n_nodes
    ceoff = c * n_edges

    def chunk(g, carry):
        base = t0 + g * _C
        pltpu.sync_copy(src_hbm.at[pl.ds(base, _C)], src_v)
        pltpu.sync_copy(dst_hbm.at[pl.ds(base, _C)], dst_v)

        def mk_idx(i, cr):
            sl = pl.ds(i * _L, _L)
            srcT_v[sl] = src_v[sl] + coff
            dstT_v[sl] = dst_v[sl] + coff
            return cr
        lax.fori_loop(0, _C // _L, mk_idx, 0)

        cp1 = pltpu.async_copy(dhc_hbm.at[srcT_v], dh_v, sem1)
        cp2 = pltpu.async_copy(ehc_hbm.at[dstT_v], eh_v, sem2)
        cp3 = pltpu.async_copy(bhc_hbm.at[srcT_v], bh_v, sem3)
        pltpu.sync_copy(cec_hbm.at[pl.ds(ceoff + base, _C)], ce_v)
        cp1.wait()
        cp2.wait()
        cp3.wait()

        def comp(i, cr):
            for j in range(G):
                sl = pl.ds(j * _L, _L)
                eij = ce_v[i, sl] + dh_v[i, sl] + eh_v[i, sl]
                ce_v[i, sl] = eij
                plsc.addupdate(st_v.at[pl.ds(j * _L, _L)], eij)
                plsc.addupdate(st_v.at[pl.ds(hh + j * _L, _L)], eij * eij)
                sg = 1.0 / (1.0 + jnp.exp(-eij))
                sg_v[i, sl] = sg
                sb_v[i, sl] = sg * bh_v[i, sl]
            return cr
        lax.fori_loop(0, _C, comp, 0)

        pltpu.sync_copy(ce_v, eij_hbm.at[pl.ds(ceoff + base, _C)])
        pltpu.sync_copy(sg_v, den_s.at[dst_v], add=True)
        pltpu.sync_copy(sb_v, num_s.at[dst_v], add=True)
        return carry
    lax.fori_loop(0, nch, chunk, 0)

    plsc.subcore_barrier()
    noff = c * n_nodes

    def flush(k, carry):
        off = base0 + k * _C
        pltpu.sync_copy(num_s.at[pl.ds(off, _C)], num_hbm.at[pl.ds(noff + off, _C)])
        pltpu.sync_copy(den_s.at[pl.ds(off, _C)], den_hbm.at[pl.ds(noff + off, _C)])
        return carry
    lax.fori_loop(0, ncopy, flush, 0)
    pltpu.sync_copy(st_v, stats_hbm.at[c, s])


def _h_out_body(h_ref, wa_ref, ba_ref, numl_ref, numr_ref, denl_ref, denr_ref,
                g_ref, b_ref, out_ref):
    hb = h_ref[...]
    ah = jnp.dot(hb, wa_ref[...], preferred_element_type=jnp.float32) + ba_ref[...]
    num = jnp.concatenate([numl_ref[...], numr_ref[...]], axis=1)
    den = jnp.concatenate([denl_ref[...], denr_ref[...]], axis=1)
    h_new = ah + num / (den + 1e-6)
    mu = jnp.mean(h_new, axis=0)
    var = jnp.mean((h_new - mu) ** 2, axis=0)
    xn = (h_new - mu) / jnp.sqrt(var + 1e-5) * g_ref[...] + b_ref[...]
    out_ref[...] = hb + jnp.maximum(xn, 0.0)


def _e_out_body(n_edges, hh, e_ref, eijl_ref, eijr_ref, st_ref, g_ref, b_ref,
                out_ref):
    st = st_ref[...]
    ssum = jnp.concatenate([jnp.sum(st[0, :, :hh], axis=0),
                            jnp.sum(st[1, :, :hh], axis=0)])
    ssq = jnp.concatenate([jnp.sum(st[0, :, hh:], axis=0),
                           jnp.sum(st[1, :, hh:], axis=0)])
    mu = ssum / n_edges
    var = ssq / n_edges - mu * mu
    eij = jnp.concatenate([eijl_ref[...], eijr_ref[...]], axis=1)
    xn = (eij - mu) / jnp.sqrt(var + 1e-5) * g_ref[...] + b_ref[...]
    out_ref[...] = e_ref[...] + jnp.maximum(xn, 0.0)


def kernel(h, edge_index, e, WA, bA, WB, bB, WC, bC, WD, bD, WE, bE,
           gamma_h, beta_h, gamma_e, beta_e):
    n, d = h.shape
    ne = e.shape[0]
    hh = d // 2
    f32 = jnp.float32

    src = edge_index[0].astype(jnp.int32)
    dst = edge_index[1].astype(jnp.int32)
    wsplit = lambda w: w.reshape(d, _NC, hh).transpose(1, 0, 2).astype(f32)
    bsplit = lambda v: v.reshape(_NC, 1, hh).astype(f32)
    brow = lambda v: v.reshape(1, d).astype(f32)

    bn = 2000
    nb = n // bn
    proj_h = pl.pallas_call(
        _proj_h_body,
        grid=(nb, _NC),
        in_specs=[
            pl.BlockSpec((bn, d), lambda i, hf: (i, 0)),
            pl.BlockSpec((1, d, hh), lambda i, hf: (hf, 0, 0)),
            pl.BlockSpec((1, d, hh), lambda i, hf: (hf, 0, 0)),
            pl.BlockSpec((1, d, hh), lambda i, hf: (hf, 0, 0)),
            pl.BlockSpec((1, 1, hh), lambda i, hf: (hf, 0, 0)),
            pl.BlockSpec((1, 1, hh), lambda i, hf: (hf, 0, 0)),
            pl.BlockSpec((1, 1, hh), lambda i, hf: (hf, 0, 0)),
        ],
        out_specs=[
            pl.BlockSpec((bn, hh), lambda i, hf: (hf * nb + i, 0)),
            pl.BlockSpec((bn, hh), lambda i, hf: (hf * nb + i, 0)),
            pl.BlockSpec((bn, hh), lambda i, hf: (hf * nb + i, 0)),
        ],
        out_shape=[
            jax.ShapeDtypeStruct((_NC * n, hh), f32),
            jax.ShapeDtypeStruct((_NC * n, hh), f32),
            jax.ShapeDtypeStruct((_NC * n, hh), f32),
        ],
    )
    bhc, dhc, ehc = proj_h(h, wsplit(WB), wsplit(WD), wsplit(WE),
                           bsplit(bB), bsplit(bD), bsplit(bE))

    be = 2000
    nbe = ne // be
    proj_e = pl.pallas_call(
        _proj_e_body,
        grid=(nbe, _NC),
        in_specs=[
            pl.BlockSpec((be, d), lambda i, hf: (i, 0)),
            pl.BlockSpec((1, d, hh), lambda i, hf: (hf, 0, 0)),
            pl.BlockSpec((1, 1, hh), lambda i, hf: (hf, 0, 0)),
        ],
        out_specs=pl.BlockSpec((be, hh), lambda i, hf: (hf * nbe + i, 0)),
        out_shape=jax.ShapeDtypeStruct((_NC * ne, hh), f32),
    )
    cec = proj_e(e, wsplit(WC), bsplit(bC))

    mesh = plsc.VectorSubcoreMesh(core_axis_name="c", subcore_axis_name="s")
    sc_edge = pl.kernel(
        functools.partial(_sc_edge_body, n, ne, hh),
        out_type=(
            jax.ShapeDtypeStruct((_NC * ne, hh), f32),
            jax.ShapeDtypeStruct((_NC * n, hh), f32),
            jax.ShapeDtypeStruct((_NC * n, hh), f32),
            jax.ShapeDtypeStruct((_NC, _NS, 2 * hh), f32),
        ),
        mesh=mesh,
        compiler_params=pltpu.CompilerParams(use_tc_tiling_on_sc=False),
        scratch_types=[
            pltpu.VMEM((_C,), jnp.int32),
            pltpu.VMEM((_C,), jnp.int32),
            pltpu.VMEM((_C,), jnp.int32),
            pltpu.VMEM((_C,), jnp.int32),
            pltpu.VMEM((_C, hh), f32),
            pltpu.VMEM((_C, hh), f32),
            pltpu.VMEM((_C, hh), f32),
            pltpu.VMEM((_C, hh), f32),
            pltpu.VMEM((_C, hh), f32),
            pltpu.VMEM((_C, hh), f32),
            pltpu.VMEM((2 * hh,), f32),
            pltpu.VMEM((_C, hh), f32),
            pltpu.VMEM_SHARED((n, hh), f32),
            pltpu.VMEM_SHARED((n, hh), f32),
            pltpu.SemaphoreType.DMA,
            pltpu.SemaphoreType.DMA,
            pltpu.SemaphoreType.DMA,
        ],
    )
    eijc, numc, denc, stats = sc_edge(src, dst, dhc, ehc, bhc, cec)

    h_out_call = pl.pallas_call(
        _h_out_body,
        grid=(1,),
        in_specs=[
            pl.BlockSpec((n, d), lambda i: (0, 0)),
            pl.BlockSpec((d, d), lambda i: (0, 0)),
            pl.BlockSpec((1, d), lambda i: (0, 0)),
            pl.BlockSpec((n, hh), lambda i: (0, 0)),
            pl.BlockSpec((n, hh), lambda i: (1, 0)),
            pl.BlockSpec((n, hh), lambda i: (0, 0)),
            pl.BlockSpec((n, hh), lambda i: (1, 0)),
            pl.BlockSpec((1, d), lambda i: (0, 0)),
            pl.BlockSpec((1, d), lambda i: (0, 0)),
        ],
        out_specs=pl.BlockSpec((n, d), lambda i: (0, 0)),
        out_shape=jax.ShapeDtypeStruct((n, d), f32),
    )
    h_out = h_out_call(h, WA.astype(f32), brow(bA), numc, numc, denc, denc,
                       brow(gamma_h), brow(beta_h))

    be2 = 4000
    nb2 = ne // be2
    e_out_call = pl.pallas_call(
        functools.partial(_e_out_body, ne, hh),
        grid=(nb2,),
        in_specs=[
            pl.BlockSpec((be2, d), lambda i: (i, 0)),
            pl.BlockSpec((be2, hh), lambda i: (i, 0)),
            pl.BlockSpec((be2, hh), lambda i: (nb2 + i, 0)),
            pl.BlockSpec((_NC, _NS, 2 * hh), lambda i: (0, 0, 0)),
            pl.BlockSpec((1, d), lambda i: (0, 0)),
            pl.BlockSpec((1, d), lambda i: (0, 0)),
        ],
        out_specs=pl.BlockSpec((be2, d), lambda i: (i, 0)),
        out_shape=jax.ShapeDtypeStruct((ne, d), f32),
    )
    e_out = e_out_call(e, eijc, eijc, stats, brow(gamma_e), brow(beta_e))

    return (h_out, e_out)


# speedup vs baseline: 2.1861x; 1.3764x over previous
import functools

import jax
import jax.numpy as jnp
from jax import lax
from jax.experimental import pallas as pl
from jax.experimental.pallas import tpu as pltpu
from jax.experimental.pallas import tpu_sc as plsc

_NC = 2
_NS = 16
_L = 16
_C = 80


def _proj_h_body(h_ref, wb, wd, we, bb, bd, be_, bh, dh, eh):
    hb = h_ref[...]
    bh[...] = jnp.dot(hb, wb[...], preferred_element_type=jnp.float32) + bb[...]
    dh[...] = jnp.dot(hb, wd[...], preferred_element_type=jnp.float32) + bd[...]
    eh[...] = jnp.dot(hb, we[...], preferred_element_type=jnp.float32) + be_[...]


def _proj_e_body(e_ref, wc, bc, ce):
    ce[...] = (jnp.dot(e_ref[...], wc[...], preferred_element_type=jnp.float32)
               + bc[...])


def _sc_edge_body(n_nodes, n_edges, hh,
                  src_hbm, dst_hbm, dhc_hbm, ehc_hbm, bhc_hbm, ce_hbm,
                  eij_hbm, num_hbm, den_hbm, stats_hbm,
                  src_v, dst_v, srcT_v, dstT_v,
                  dh_v, eh_v, bh_v, ce_v, eij_v, sg_v, sb_v,
                  st_v, zb_v, num_s, den_s,
                  sem1, sem2, sem3):
    c = lax.axis_index("c")
    s = lax.axis_index("s")
    G = hh // _L

    zero = jnp.zeros((_L,), jnp.float32)

    def zb_row(i, carry):
        for j in range(G):
            zb_v[i, pl.ds(j * _L, _L)] = zero
        return carry
    lax.fori_loop(0, _C, zb_row, 0)

    for j in range(2 * G):
        st_v[pl.ds(j * _L, _L)] = zero

    rpt = (-(-n_nodes // _NS) + _C - 1) // _C * _C
    base0 = s * rpt
    nrows = jnp.maximum(0, jnp.minimum(rpt, n_nodes - base0))
    ncopy = nrows // _C

    def z_copy(k, carry):
        off = base0 + k * _C
        pltpu.sync_copy(zb_v, num_s.at[pl.ds(off, _C)])
        pltpu.sync_copy(zb_v, den_s.at[pl.ds(off, _C)])
        return carry
    lax.fori_loop(0, ncopy, z_copy, 0)
    plsc.subcore_barrier()

    et = n_edges // _NS
    nch = et // _C
    t0 = s * et

    def chunk(g, carry):
        base = t0 + g * _C
        pltpu.sync_copy(src_hbm.at[pl.ds(base, _C)], src_v)
        pltpu.sync_copy(dst_hbm.at[pl.ds(base, _C)], dst_v)

        def mk_idx(i, cr):
            sl = pl.ds(i * _L, _L)
            srcT_v[sl] = src_v[sl] * 2 + c
            dstT_v[sl] = dst_v[sl] * 2 + c
            return cr
        lax.fori_loop(0, _C // _L, mk_idx, 0)

        cp1 = pltpu.async_copy(dhc_hbm.at[srcT_v], dh_v, sem1)
        cp2 = pltpu.async_copy(ehc_hbm.at[dstT_v], eh_v, sem2)
        cp3 = pltpu.async_copy(bhc_hbm.at[srcT_v], bh_v, sem3)
        pltpu.sync_copy(ce_hbm.at[pl.ds(base, _C), pl.ds(c * hh, hh)], ce_v)
        cp1.wait()
        cp2.wait()
        cp3.wait()

        def comp(i, cr):
            for j in range(G):
                sl = pl.ds(j * _L, _L)
                eij = ce_v[i, sl] + dh_v[i, sl] + eh_v[i, sl]
                eij_v[i, sl] = eij
                plsc.addupdate(st_v.at[pl.ds(j * _L, _L)], eij)
                plsc.addupdate(st_v.at[pl.ds(hh + j * _L, _L)], eij * eij)
                sg = 1.0 / (1.0 + jnp.exp(-eij))
                sg_v[i, sl] = sg
                sb_v[i, sl] = sg * bh_v[i, sl]
            return cr
        lax.fori_loop(0, _C, comp, 0)

        pltpu.sync_copy(eij_v, eij_hbm.at[pl.ds(base, _C), pl.ds(c * hh, hh)])
        pltpu.sync_copy(sg_v, den_s.at[dst_v], add=True)
        pltpu.sync_copy(sb_v, num_s.at[dst_v], add=True)
        return carry
    lax.fori_loop(0, nch, chunk, 0)

    plsc.subcore_barrier()
    noff = c * n_nodes

    def flush(k, carry):
        off = base0 + k * _C
        pltpu.sync_copy(num_s.at[pl.ds(off, _C)], num_hbm.at[pl.ds(noff + off, _C)])
        pltpu.sync_copy(den_s.at[pl.ds(off, _C)], den_hbm.at[pl.ds(noff + off, _C)])
        return carry
    lax.fori_loop(0, ncopy, flush, 0)
    pltpu.sync_copy(st_v, stats_hbm.at[c, s])


def _h_out_body(h_ref, wa_ref, ba_ref, numl_ref, numr_ref, denl_ref, denr_ref,
                g_ref, b_ref, out_ref):
    hb = h_ref[...]
    ah = jnp.dot(hb, wa_ref[...], preferred_element_type=jnp.float32) + ba_ref[...]
    num = jnp.concatenate([numl_ref[...], numr_ref[...]], axis=1)
    den = jnp.concatenate([denl_ref[...], denr_ref[...]], axis=1)
    h_new = ah + num / (den + 1e-6)
    mu = jnp.mean(h_new, axis=0)
    var = jnp.mean((h_new - mu) ** 2, axis=0)
    xn = (h_new - mu) / jnp.sqrt(var + 1e-5) * g_ref[...] + b_ref[...]
    out_ref[...] = hb + jnp.maximum(xn, 0.0)


def _e_out_body(n_edges, hh, e_ref, eij_ref, st_ref, g_ref, b_ref, out_ref):
    st = st_ref[...]
    ssum = jnp.concatenate([jnp.sum(st[0, :, :hh], axis=0),
                            jnp.sum(st[1, :, :hh], axis=0)])
    ssq = jnp.concatenate([jnp.sum(st[0, :, hh:], axis=0),
                           jnp.sum(st[1, :, hh:], axis=0)])
    mu = ssum / n_edges
    var = ssq / n_edges - mu * mu
    eij = eij_ref[...]
    xn = (eij - mu) / jnp.sqrt(var + 1e-5) * g_ref[...] + b_ref[...]
    out_ref[...] = e_ref[...] + jnp.maximum(xn, 0.0)


def kernel(h, edge_index, e, WA, bA, WB, bB, WC, bC, WD, bD, WE, bE,
           gamma_h, beta_h, gamma_e, beta_e):
    n, d = h.shape
    ne = e.shape[0]
    hh = d // 2
    f32 = jnp.float32

    src = edge_index[0].astype(jnp.int32)
    dst = edge_index[1].astype(jnp.int32)
    brow = lambda v: v.reshape(1, d).astype(f32)

    bn = 2000
    nb = n // bn
    full = lambda i: (i, 0)
    w_spec = pl.BlockSpec((d, d), lambda i: (0, 0))
    b_spec = pl.BlockSpec((1, d), lambda i: (0, 0))
    proj_h = pl.pallas_call(
        _proj_h_body,
        grid=(nb,),
        in_specs=[pl.BlockSpec((bn, d), full),
                  w_spec, w_spec, w_spec, b_spec, b_spec, b_spec],
        out_specs=[pl.BlockSpec((bn, d), full)] * 3,
        out_shape=[jax.ShapeDtypeStruct((n, d), f32)] * 3,
    )
    bh_t, dh_t, eh_t = proj_h(h, WB.astype(f32), WD.astype(f32),
                              WE.astype(f32), brow(bB), brow(bD), brow(bE))

    be = 2000
    nbe = ne // be
    proj_e = pl.pallas_call(
        _proj_e_body,
        grid=(nbe,),
        in_specs=[pl.BlockSpec((be, d), full), w_spec, b_spec],
        out_specs=pl.BlockSpec((be, d), full),
        out_shape=jax.ShapeDtypeStruct((ne, d), f32),
    )
    ce = proj_e(e, WC.astype(f32), brow(bC))

    mesh = plsc.VectorSubcoreMesh(core_axis_name="c", subcore_axis_name="s")
    sc_edge = pl.kernel(
        functools.partial(_sc_edge_body, n, ne, hh),
        out_type=(
            jax.ShapeDtypeStruct((ne, d), f32),
            jax.ShapeDtypeStruct((_NC * n, hh), f32),
            jax.ShapeDtypeStruct((_NC * n, hh), f32),
            jax.ShapeDtypeStruct((_NC, _NS, 2 * hh), f32),
        ),
        mesh=mesh,
        compiler_params=pltpu.CompilerParams(use_tc_tiling_on_sc=False),
        scratch_types=[
            pltpu.VMEM((_C,), jnp.int32),
            pltpu.VMEM((_C,), jnp.int32),
            pltpu.VMEM((_C,), jnp.int32),
            pltpu.VMEM((_C,), jnp.int32),
            pltpu.VMEM((_C, hh), f32),
            pltpu.VMEM((_C, hh), f32),
            pltpu.VMEM((_C, hh), f32),
            pltpu.VMEM((_C, hh), f32),
            pltpu.VMEM((_C, hh), f32),
            pltpu.VMEM((_C, hh), f32),
            pltpu.VMEM((_C, hh), f32),
            pltpu.VMEM((2 * hh,), f32),
            pltpu.VMEM((_C, hh), f32),
            pltpu.VMEM_SHARED((n, hh), f32),
            pltpu.VMEM_SHARED((n, hh), f32),
            pltpu.SemaphoreType.DMA,
            pltpu.SemaphoreType.DMA,
            pltpu.SemaphoreType.DMA,
        ],
    )
    eij, numc, denc, stats = sc_edge(src, dst,
                                     dh_t.reshape(_NC * n, hh),
                                     eh_t.reshape(_NC * n, hh),
                                     bh_t.reshape(_NC * n, hh),
                                     ce)

    h_out_call = pl.pallas_call(
        _h_out_body,
        grid=(1,),
        in_specs=[
            pl.BlockSpec((n, d), lambda i: (0, 0)),
            pl.BlockSpec((d, d), lambda i: (0, 0)),
            pl.BlockSpec((1, d), lambda i: (0, 0)),
            pl.BlockSpec((n, hh), lambda i: (0, 0)),
            pl.BlockSpec((n, hh), lambda i: (1, 0)),
            pl.BlockSpec((n, hh), lambda i: (0, 0)),
            pl.BlockSpec((n, hh), lambda i: (1, 0)),
            pl.BlockSpec((1, d), lambda i: (0, 0)),
            pl.BlockSpec((1, d), lambda i: (0, 0)),
        ],
        out_specs=pl.BlockSpec((n, d), lambda i: (0, 0)),
        out_shape=jax.ShapeDtypeStruct((n, d), f32),
    )
    h_out = h_out_call(h, WA.astype(f32), brow(bA), numc, numc, denc, denc,
                       brow(gamma_h), brow(beta_h))

    be2 = 4000
    nb2 = ne // be2
    e_out_call = pl.pallas_call(
        functools.partial(_e_out_body, ne, hh),
        grid=(nb2,),
        in_specs=[
            pl.BlockSpec((be2, d), full),
            pl.BlockSpec((be2, d), full),
            pl.BlockSpec((_NC, _NS, 2 * hh), lambda i: (0, 0, 0)),
            pl.BlockSpec((1, d), lambda i: (0, 0)),
            pl.BlockSpec((1, d), lambda i: (0, 0)),
        ],
        out_specs=pl.BlockSpec((be2, d), full),
        out_shape=jax.ShapeDtypeStruct((ne, d), f32),
    )
    e_out = e_out_call(e, eij, stats, brow(gamma_e), brow(beta_e))

    return (h_out, e_out)
